# baseline (speedup 1.0000x reference)
import functools

import jax
import jax.numpy as jnp
from jax import lax
from jax.experimental import pallas as pl
from jax.experimental.pallas import tpu as pltpu
from jax.experimental.pallas import tpu_sc as plsc

E = 8
TOP_K = 2
T = 2048
HS = 1024
DFF = 1024
TK = T * TOP_K
TB = 256
SB = TB * TOP_K
BR = 256
NPAD = TK + E * BR
NBLK = NPAD // BR
NW = 32
RPW = NPAD // NW
CH = 96
TPW = T // NW
CT = 32


def _router_body(x_ref, rw_ref, pos_ref, word_ref, blk_ref, carry, eo, ro):
    b = pl.program_id(0)

    @pl.when(b == 0)
    def _init():
        carry[...] = jnp.zeros_like(carry)

    xb = x_ref[...]
    logits = jnp.dot(xb, rw_ref[...], preferred_element_type=jnp.float32)
    scores = jax.nn.softmax(logits, axis=-1)
    lane = lax.broadcasted_iota(jnp.int32, (TB, E), 1)
    m1 = jnp.max(scores, axis=-1, keepdims=True)
    i1 = jnp.min(jnp.where(scores == m1, lane, E), axis=-1, keepdims=True)
    scores2 = jnp.where(lane == i1, -jnp.inf, scores)
    m2 = jnp.max(scores2, axis=-1, keepdims=True)
    i2 = jnp.min(jnp.where(scores2 == m2, lane, E), axis=-1, keepdims=True)

    e_s = jnp.concatenate([i1, i2], axis=0)
    w_s = jnp.concatenate([m1, m2], axis=0)
    lane_s = lax.broadcasted_iota(jnp.int32, (SB, E), 1)
    onehot = (lane_s == e_s).astype(jnp.float32)
    row = lax.broadcasted_iota(jnp.int32, (SB, SB), 0)
    col = lax.broadcasted_iota(jnp.int32, (SB, SB), 1)
    tril = (col < row).astype(jnp.float32)
    within = jnp.dot(tril, onehot, preferred_element_type=jnp.float32)
    within = within + carry[...]
    rank = jnp.sum(onehot * within, axis=-1, keepdims=True)
    carry[...] += jnp.sum(onehot, axis=0, keepdims=True)

    eo[pl.ds(b * SB, SB), :] = e_s.astype(jnp.float32)
    ro[pl.ds(b * SB, SB), :] = rank
    word_ref[pl.ds(b * SB, SB), :] = w_s

    @pl.when(b == E - 1)
    def _finalize():
        hist = carry[...]
        pc = jnp.floor((hist + (BR - 1.0)) / BR) * BR
        e8 = lax.broadcasted_iota(jnp.int32, (1, E), 1).astype(jnp.float32)
        e_all = eo[...]
        posf = ro[...]
        for ex in range(E):
            start = jnp.sum(pc * (e8 < ex))
            posf = posf + jnp.where(e_all == ex, start, 0.0)
        pos_ref[...] = posf.astype(jnp.int32)
        bstart = (lax.broadcasted_iota(jnp.int32, (1, 32), 1)
                  .astype(jnp.float32) * BR)
        acc = jnp.zeros((1, 32), jnp.float32)
        for ex in range(E):
            end = jnp.sum(pc * (e8 <= ex))
            acc = acc + jnp.where(bstart >= end, 1.0, 0.0)
        blk_ref[...] = jnp.minimum(acc, E - 1.0).astype(jnp.int32)


def _run_router(xf, router_w):
    return pl.pallas_call(
        _router_body,
        grid=(E,),
        in_specs=[
            pl.BlockSpec((TB, HS), lambda b: (b, 0)),
            pl.BlockSpec((HS, E), lambda b: (0, 0)),
        ],
        out_specs=[
            pl.BlockSpec((TK, 1), lambda b: (0, 0)),
            pl.BlockSpec((TK, 1), lambda b: (0, 0)),
            pl.BlockSpec((1, 32), lambda b: (0, 0)),
        ],
        out_shape=[
            jax.ShapeDtypeStruct((TK, 1), jnp.int32),
            jax.ShapeDtypeStruct((TK, 1), jnp.float32),
            jax.ShapeDtypeStruct((1, 32), jnp.int32),
        ],
        scratch_shapes=[
            pltpu.VMEM((1, E), jnp.float32),
            pltpu.VMEM((TK, 1), jnp.float32),
            pltpu.VMEM((TK, 1), jnp.float32),
        ],
    )(xf, router_w)


@functools.lru_cache(maxsize=None)
def _make_gather_kernel():
    mesh = plsc.VectorSubcoreMesh(core_axis_name="c", subcore_axis_name="s")
    return pl.kernel(
        _gather_body,
        out_type=jax.ShapeDtypeStruct((NPAD, HS), jnp.float32),
        mesh=mesh,
        scratch_types=[
            pltpu.VMEM((TK,), jnp.int32),
            pltpu.VMEM((CH,), jnp.int32),
            pltpu.VMEM((CH,), jnp.int32),
            pltpu.VMEM((CH, HS), jnp.float32),
            pltpu.SemaphoreType.DMA,
        ],
        compiler_params=pltpu.CompilerParams(needs_layout_passes=False),
    )


def _gather_body(pos_hbm, x_hbm, xg_hbm, pos_v, src0, src1, rows_v, sem):
    wid = lax.axis_index("s") * 2 + lax.axis_index("c")
    pltpu.sync_copy(pos_hbm, pos_v)
    base = wid * RPW
    srcs = (src0, src1)
    for b in range(RPW // CH):
        for i in range(CH // 16):
            srcs[b][pl.ds(i * 16, 16)] = jnp.zeros((16,), jnp.int32)

    def scat_body(i, c):
        s16 = i * 16 + lax.iota(jnp.int32, 16)
        pv = pos_v[pl.ds(i * 16, 16)]
        tok = ((s16 >> 9) << 8) | (s16 & 255)
        for b in range(RPW // CH):
            rel = pv - (base + b * CH)
            m = (rel >= 0) & (rel < CH)
            plsc.store_scatter(srcs[b], [jnp.where(m, rel, 0)], tok, mask=m)
        return c

    lax.fori_loop(0, TK // 16, scat_body, 0)

    for b in range(RPW // CH):
        pltpu.async_copy(x_hbm.at[srcs[b]], rows_v, sem).wait()
        pltpu.sync_copy(rows_v, xg_hbm.at[pl.ds(base + b * CH, CH)])


def _gemm_body(be_ref, xg_ref, w1_ref, w2_ref, z_ref):
    h = jnp.dot(xg_ref[...], w1_ref[0], preferred_element_type=jnp.float32)
    z_ref[...] = jnp.dot(h, w2_ref[0], preferred_element_type=jnp.float32)


def _run_gemm(blk_e, xg, W1, W2):
    grid_spec = pltpu.PrefetchScalarGridSpec(
        num_scalar_prefetch=1,
        grid=(NBLK,),
        in_specs=[
            pl.BlockSpec((BR, HS), lambda b, be: (b, 0)),
            pl.BlockSpec((1, HS, DFF), lambda b, be: (be[b], 0, 0)),
            pl.BlockSpec((1, DFF, HS), lambda b, be: (be[b], 0, 0)),
        ],
        out_specs=pl.BlockSpec((BR, HS), lambda b, be: (b, 0)),
    )
    return pl.pallas_call(
        _gemm_body,
        grid_spec=grid_spec,
        out_shape=jax.ShapeDtypeStruct((NPAD, HS), jnp.float32),
    )(blk_e, xg, W1, W2)


@functools.lru_cache(maxsize=None)
def _make_combine_kernel():
    mesh = plsc.VectorSubcoreMesh(core_axis_name="c", subcore_axis_name="s")
    return pl.kernel(
        _combine_body,
        out_type=jax.ShapeDtypeStruct((T, HS), jnp.float32),
        mesh=mesh,
        scratch_types=[
            pltpu.VMEM((CT,), jnp.int32),
            pltpu.VMEM((CT,), jnp.int32),
            pltpu.VMEM((CT,), jnp.float32),
            pltpu.VMEM((CT,), jnp.float32),
            pltpu.VMEM((CT, HS), jnp.float32),
            pltpu.VMEM((CT, HS), jnp.float32),
            pltpu.VMEM((CT, HS), jnp.float32),
            pltpu.SemaphoreType.DMA,
        ],
        compiler_params=pltpu.CompilerParams(needs_layout_passes=False),
    )


def _combine_body(pos_hbm, word_hbm, z_hbm, out_hbm,
                  idx0, idx1, w0v, w1v, rows0, rows1, outb, sem):
    wid = lax.axis_index("s") * 2 + lax.axis_index("c")
    t0 = wid * TPW
    s0 = ((t0 >> 8) << 9) | (t0 & 255)
    for c in range(TPW // CT):
        sc0 = pl.multiple_of(s0 + c * CT, 32)
        pltpu.sync_copy(pos_hbm.at[pl.ds(sc0, CT)], idx0)
        pltpu.sync_copy(pos_hbm.at[pl.ds(sc0 + TB, CT)], idx1)
        pltpu.sync_copy(word_hbm.at[pl.ds(sc0, CT)], w0v)
        pltpu.sync_copy(word_hbm.at[pl.ds(sc0 + TB, CT)], w1v)
        pltpu.async_copy(z_hbm.at[idx0], rows0, sem).wait()
        pltpu.async_copy(z_hbm.at[idx1], rows1, sem).wait()

        def row_body(r, cc):
            bidx = jnp.full((16,), r, jnp.int32)
            wa = plsc.load_gather(w0v, [bidx])
            wb = plsc.load_gather(w1v, [bidx])
            for u in range(HS // 16):
                a = rows0[r, pl.ds(u * 16, 16)]
                bb = rows1[r, pl.ds(u * 16, 16)]
                outb[r, pl.ds(u * 16, 16)] = wa * a + wb * bb
            return cc

        lax.fori_loop(0, CT, row_body, 0)
        pltpu.sync_copy(outb, out_hbm.at[pl.ds(t0 + c * CT, CT)])


def kernel(x, router_w, W1, W2):
    sl, bs, hs = x.shape
    xf = x.reshape(T, HS)
    pos2, word2, blk2 = _run_router(xf, router_w)
    pos = pos2.reshape(TK)
    word = word2.reshape(TK)
    blk_e = blk2.reshape(32)[:NBLK]
    xg = _make_gather_kernel()(pos, xf)
    z = _run_gemm(blk_e, xg, W1, W2)
    out = _make_combine_kernel()(pos, word, z)
    return out.reshape(sl, bs, hs)
